# Initial kernel scaffold; baseline (speedup 1.0000x reference)
import jax
import jax.numpy as jnp
from jax.experimental import pallas as pl


def kernel(x_swin, x_conv, noise_eps, params):
    raise NotImplementedError("write your pallas kernel here")


# speedup vs baseline: 1.7762x; 1.7762x over previous
import functools

import jax
import jax.numpy as jnp
import numpy as np
from jax.experimental import pallas as pl
from jax.experimental.pallas import tpu as pltpu

B, S, D, H, E, K = 1, 2048, 1024, 16, 8, 2
HID = 4 * D
DH = D // H
HBLK = 1024
NH = HID // HBLK


def _ln(x, g, b):
    m = jnp.mean(x, axis=-1, keepdims=True)
    v = jnp.mean((x - m) ** 2, axis=-1, keepdims=True)
    return (x - m) * jax.lax.rsqrt(v + 1e-5) * g + b


def _ln2_kernel(xs_ref, xc_ref, gs_ref, bs_ref, gc_ref, bc_ref,
                xs_n_ref, xc_n_ref):
    xs_n_ref[...] = _ln(xs_ref[...], gs_ref[...], bs_ref[...])
    xc_n_ref[...] = _ln(xc_ref[...], gc_ref[...], bc_ref[...])


def _ln2(xs, xc, gs, bs, gc, bc):
    return pl.pallas_call(
        _ln2_kernel,
        out_shape=(jax.ShapeDtypeStruct((S, D), jnp.float32),
                   jax.ShapeDtypeStruct((S, D), jnp.float32)),
    )(xs, xc, gs.reshape(1, D), bs.reshape(1, D), gc.reshape(1, D),
      bc.reshape(1, D))


def _mha_kernel(qn_ref, kvn_ref, wq_ref, wk_ref, wv_ref, bq_ref, bk_ref,
                bv_ref, wo_ref, bo_ref, res_ref, out_ref):
    h = pl.program_id(0)
    nt = (((1,), (1,)), ((), ()))
    q = jax.lax.dot_general(qn_ref[...], wq_ref[...], nt,
                            preferred_element_type=jnp.float32) + bq_ref[0]
    k = jax.lax.dot_general(kvn_ref[...], wk_ref[...], nt,
                            preferred_element_type=jnp.float32) + bk_ref[0]
    v = jax.lax.dot_general(kvn_ref[...], wv_ref[...], nt,
                            preferred_element_type=jnp.float32) + bv_ref[0]
    s = jax.lax.dot_general(q, k, nt, preferred_element_type=jnp.float32)
    s = s * (1.0 / np.sqrt(DH))
    att = jax.nn.softmax(s, axis=-1)
    o = jnp.dot(att, v, preferred_element_type=jnp.float32)
    part = jax.lax.dot_general(o, wo_ref[0], nt,
                               preferred_element_type=jnp.float32)

    @pl.when(h == 0)
    def _():
        out_ref[...] = res_ref[...] + bo_ref[...]

    out_ref[...] += part


def _mha(qn, kvn, in_w, in_b, out_w, out_b, res):
    in_b2 = in_b.reshape(3 * H, 1, DH)
    grid = (H,)
    return pl.pallas_call(
        _mha_kernel,
        grid=grid,
        in_specs=[
            pl.BlockSpec((S, D), lambda h: (0, 0)),
            pl.BlockSpec((S, D), lambda h: (0, 0)),
            pl.BlockSpec((DH, D), lambda h: (h, 0)),
            pl.BlockSpec((DH, D), lambda h: (H + h, 0)),
            pl.BlockSpec((DH, D), lambda h: (2 * H + h, 0)),
            pl.BlockSpec((1, 1, DH), lambda h: (h, 0, 0)),
            pl.BlockSpec((1, 1, DH), lambda h: (H + h, 0, 0)),
            pl.BlockSpec((1, 1, DH), lambda h: (2 * H + h, 0, 0)),
            pl.BlockSpec((1, D, DH), lambda h: (h, 0, 0)),
            pl.BlockSpec((1, D), lambda h: (0, 0)),
            pl.BlockSpec((S, D), lambda h: (0, 0)),
        ],
        out_specs=pl.BlockSpec((S, D), lambda h: (0, 0)),
        out_shape=jax.ShapeDtypeStruct((S, D), jnp.float32),
    )(qn, kvn, in_w, in_w, in_w, in_b2, in_b2, in_b2,
      out_w.reshape(D, H, DH).transpose(1, 0, 2), out_b.reshape(1, D), res)


def _moe_prep_kernel(xs_ref, xc_ref, g_ref, b_ref, rw_ref, rb_ref,
                     x_ref, hn_ref, lg_ref):
    x = xs_ref[...] + xc_ref[...]
    hn = _ln(x, g_ref[...], b_ref[...])
    x_ref[...] = x
    hn_ref[...] = hn
    nt = (((1,), (1,)), ((), ()))
    lg_ref[...] = jax.lax.dot_general(
        hn, rw_ref[...], nt, preferred_element_type=jnp.float32) + rb_ref[...]


def _moe_prep(xs, xc, g, b, rw, rb):
    return pl.pallas_call(
        _moe_prep_kernel,
        out_shape=(jax.ShapeDtypeStruct((S, D), jnp.float32),
                   jax.ShapeDtypeStruct((S, D), jnp.float32),
                   jax.ShapeDtypeStruct((S, 2 * E), jnp.float32)),
    )(xs, xc, g.reshape(1, D), b.reshape(1, D), rw, rb.reshape(1, 2 * E))


def _expert_kernel(idx_ref, probs_ref, hn_ref, x_ref, fc1_ref, b1_ref,
                   fc2_ref, b2_ref, out_ref):
    ki = pl.program_id(0)
    j = pl.program_id(1)
    p = probs_ref[0, ki]
    nt = (((1,), (1,)), ((), ()))
    h1 = jax.lax.dot_general(hn_ref[...], fc1_ref[0], nt,
                             preferred_element_type=jnp.float32) + b1_ref[0]
    h1 = h1 * 0.5 * (1.0 + jax.lax.erf(h1 * np.float32(1.0 / np.sqrt(2.0))))
    part = jax.lax.dot_general(h1, fc2_ref[0], nt,
                               preferred_element_type=jnp.float32)

    @pl.when((ki == 0) & (j == 0))
    def _():
        out_ref[...] = x_ref[...]

    @pl.when(j == 0)
    def _():
        out_ref[...] += p * b2_ref[0]

    out_ref[...] += p * part


def _experts(idx, probs, hn, x, fc1, b1, fc2, b2):
    grid = (K, NH)
    return pl.pallas_call(
        _expert_kernel,
        grid_spec=pltpu.PrefetchScalarGridSpec(
            num_scalar_prefetch=1,
            grid=grid,
            in_specs=[
                pl.BlockSpec(memory_space=pltpu.SMEM),
                pl.BlockSpec((S, D), lambda k, j, idx: (0, 0)),
                pl.BlockSpec((S, D), lambda k, j, idx: (0, 0)),
                pl.BlockSpec((1, HBLK, D), lambda k, j, idx: (idx[k], j, 0)),
                pl.BlockSpec((1, 1, HBLK), lambda k, j, idx: (idx[k], 0, j)),
                pl.BlockSpec((1, D, HBLK), lambda k, j, idx: (idx[k], 0, j)),
                pl.BlockSpec((1, 1, D), lambda k, j, idx: (idx[k], 0, 0)),
            ],
            out_specs=pl.BlockSpec((S, D), lambda k, j, idx: (0, 0)),
        ),
        out_shape=jax.ShapeDtypeStruct((S, D), jnp.float32),
    )(idx, probs, hn, x, fc1, b1, fc2, b2)


def kernel(x_swin, x_conv, noise_eps, params):
    p = params
    xs0 = x_swin.reshape(S, D)
    xc0 = x_conv.reshape(S, D)
    eps = noise_eps.reshape(S, E)

    xs_n, xc_n = _ln2(xs0, xc0, p['swin_pre_ln_g'], p['swin_pre_ln_b'],
                      p['conv_pre_ln_g'], p['conv_pre_ln_b'])
    xs = _mha(xs_n, xc_n, p['cross_swin_in_w'], p['cross_swin_in_b'],
              p['cross_swin_out_w'], p['cross_swin_out_b'], xs0)
    xc = _mha(xc_n, xs_n, p['cross_conv_in_w'], p['cross_conv_in_b'],
              p['cross_conv_out_w'], p['cross_conv_out_b'], xc0)
    ts, tc = _ln2(xs, xc, p['swin_self_ln_g'], p['swin_self_ln_b'],
                  p['conv_self_ln_g'], p['conv_self_ln_b'])
    xs = _mha(ts, ts, p['self_swin_in_w'], p['self_swin_in_b'],
              p['self_swin_out_w'], p['self_swin_out_b'], xs)
    xc = _mha(tc, tc, p['self_conv_in_w'], p['self_conv_in_b'],
              p['self_conv_out_w'], p['self_conv_out_b'], xc)

    rw = jnp.concatenate([p['router_w'], p['noise_w']], axis=0)
    rb = jnp.concatenate([p['router_b'], p['noise_b']], axis=0)
    x, hn, lg = _moe_prep(xs, xc, p['moe_ln_g'], p['moe_ln_b'], rw, rb)

    noisy = lg[:, :E] + eps * jax.nn.softplus(lg[:, E:])
    noisy = noisy.mean(axis=0)
    vals, idx = jax.lax.top_k(noisy, K)
    probs = jax.nn.softmax(vals).reshape(1, K)

    out = _experts(idx.astype(jnp.int32), probs, hn, x,
                   p['exp_fc1_w'], p['exp_fc1_b'].reshape(E, 1, HID),
                   p['exp_fc2_w'], p['exp_fc2_b'].reshape(E, 1, D))
    return out.reshape(B, S, D)


# speedup vs baseline: 1.8275x; 1.0289x over previous
import functools

import jax
import jax.numpy as jnp
import numpy as np
from jax.experimental import pallas as pl
from jax.experimental.pallas import tpu as pltpu

B, S, D, H, E, K = 1, 2048, 1024, 16, 8, 2
HID = 4 * D
DH = D // H
HBLK = 1024
NH = HID // HBLK


def _ln(x, g, b):
    m = jnp.mean(x, axis=-1, keepdims=True)
    v = jnp.mean((x - m) ** 2, axis=-1, keepdims=True)
    return (x - m) * jax.lax.rsqrt(v + 1e-5) * g + b


def _ln2_kernel(xs_ref, xc_ref, gs_ref, bs_ref, gc_ref, bc_ref,
                xs_n_ref, xc_n_ref):
    xs_n_ref[...] = _ln(xs_ref[...], gs_ref[...], bs_ref[...])
    xc_n_ref[...] = _ln(xc_ref[...], gc_ref[...], bc_ref[...])


def _ln2(xs, xc, gs, bs, gc, bc):
    return pl.pallas_call(
        _ln2_kernel,
        out_shape=(jax.ShapeDtypeStruct((S, D), jnp.float32),
                   jax.ShapeDtypeStruct((S, D), jnp.float32)),
    )(xs, xc, gs.reshape(1, D), bs.reshape(1, D), gc.reshape(1, D),
      bc.reshape(1, D))


def _mha_kernel(qn_ref, kvn_ref, wq_ref, wk_ref, wv_ref, bq_ref, bk_ref,
                bv_ref, wo_ref, bo_ref, res_ref, out_ref):
    h = pl.program_id(0)
    nt = (((1,), (1,)), ((), ()))
    bf = jnp.bfloat16
    qn = qn_ref[...].astype(bf)
    kvn = kvn_ref[...].astype(bf)
    q = jax.lax.dot_general(qn, wq_ref[...].astype(bf), nt,
                            preferred_element_type=jnp.float32) + bq_ref[0]
    k = jax.lax.dot_general(kvn, wk_ref[...].astype(bf), nt,
                            preferred_element_type=jnp.float32) + bk_ref[0]
    v = jax.lax.dot_general(kvn, wv_ref[...].astype(bf), nt,
                            preferred_element_type=jnp.float32) + bv_ref[0]
    qs = (q * (1.0 / np.sqrt(DH))).astype(bf)
    s = jax.lax.dot_general(qs, k.astype(bf), nt,
                            preferred_element_type=jnp.float32)
    att = jax.nn.softmax(s, axis=-1)
    o = jnp.dot(att.astype(bf), v.astype(bf),
                preferred_element_type=jnp.float32)
    part = jax.lax.dot_general(o.astype(bf), wo_ref[0].astype(bf), nt,
                               preferred_element_type=jnp.float32)

    @pl.when(h == 0)
    def _():
        out_ref[...] = res_ref[...] + bo_ref[...]

    out_ref[...] += part


def _mha(qn, kvn, in_w, in_b, out_w, out_b, res):
    in_b2 = in_b.reshape(3 * H, 1, DH)
    grid = (H,)
    return pl.pallas_call(
        _mha_kernel,
        grid=grid,
        in_specs=[
            pl.BlockSpec((S, D), lambda h: (0, 0)),
            pl.BlockSpec((S, D), lambda h: (0, 0)),
            pl.BlockSpec((DH, D), lambda h: (h, 0)),
            pl.BlockSpec((DH, D), lambda h: (H + h, 0)),
            pl.BlockSpec((DH, D), lambda h: (2 * H + h, 0)),
            pl.BlockSpec((1, 1, DH), lambda h: (h, 0, 0)),
            pl.BlockSpec((1, 1, DH), lambda h: (H + h, 0, 0)),
            pl.BlockSpec((1, 1, DH), lambda h: (2 * H + h, 0, 0)),
            pl.BlockSpec((1, D, DH), lambda h: (h, 0, 0)),
            pl.BlockSpec((1, D), lambda h: (0, 0)),
            pl.BlockSpec((S, D), lambda h: (0, 0)),
        ],
        out_specs=pl.BlockSpec((S, D), lambda h: (0, 0)),
        out_shape=jax.ShapeDtypeStruct((S, D), jnp.float32),
    )(qn, kvn, in_w, in_w, in_w, in_b2, in_b2, in_b2,
      out_w.reshape(D, H, DH).transpose(1, 0, 2), out_b.reshape(1, D), res)


def _moe_prep_kernel(xs_ref, xc_ref, g_ref, b_ref, rw_ref, rb_ref,
                     x_ref, hn_ref, lg_ref):
    x = xs_ref[...] + xc_ref[...]
    hn = _ln(x, g_ref[...], b_ref[...])
    x_ref[...] = x
    hn_ref[...] = hn
    nt = (((1,), (1,)), ((), ()))
    lg_ref[...] = jax.lax.dot_general(
        hn, rw_ref[...], nt, preferred_element_type=jnp.float32) + rb_ref[...]


def _moe_prep(xs, xc, g, b, rw, rb):
    return pl.pallas_call(
        _moe_prep_kernel,
        out_shape=(jax.ShapeDtypeStruct((S, D), jnp.float32),
                   jax.ShapeDtypeStruct((S, D), jnp.float32),
                   jax.ShapeDtypeStruct((S, 2 * E), jnp.float32)),
    )(xs, xc, g.reshape(1, D), b.reshape(1, D), rw, rb.reshape(1, 2 * E))


def _expert_kernel(idx_ref, probs_ref, hn_ref, x_ref, fc1_ref, b1_ref,
                   fc2_ref, b2_ref, out_ref):
    ki = pl.program_id(0)
    j = pl.program_id(1)
    p = probs_ref[0, ki]
    nt = (((1,), (1,)), ((), ()))
    bf = jnp.bfloat16
    h1 = jax.lax.dot_general(hn_ref[...].astype(bf), fc1_ref[0].astype(bf), nt,
                             preferred_element_type=jnp.float32) + b1_ref[0]
    h1 = h1 * 0.5 * (1.0 + jax.lax.erf(h1 * np.float32(1.0 / np.sqrt(2.0))))
    part = jax.lax.dot_general(h1.astype(bf), fc2_ref[0].astype(bf), nt,
                               preferred_element_type=jnp.float32)

    @pl.when((ki == 0) & (j == 0))
    def _():
        out_ref[...] = x_ref[...]

    @pl.when(j == 0)
    def _():
        out_ref[...] += p * b2_ref[0]

    out_ref[...] += p * part


def _experts(idx, probs, hn, x, fc1, b1, fc2, b2):
    grid = (K, NH)
    return pl.pallas_call(
        _expert_kernel,
        grid_spec=pltpu.PrefetchScalarGridSpec(
            num_scalar_prefetch=1,
            grid=grid,
            in_specs=[
                pl.BlockSpec(memory_space=pltpu.SMEM),
                pl.BlockSpec((S, D), lambda k, j, idx: (0, 0)),
                pl.BlockSpec((S, D), lambda k, j, idx: (0, 0)),
                pl.BlockSpec((1, HBLK, D), lambda k, j, idx: (idx[k], j, 0)),
                pl.BlockSpec((1, 1, HBLK), lambda k, j, idx: (idx[k], 0, j)),
                pl.BlockSpec((1, D, HBLK), lambda k, j, idx: (idx[k], 0, j)),
                pl.BlockSpec((1, 1, D), lambda k, j, idx: (idx[k], 0, 0)),
            ],
            out_specs=pl.BlockSpec((S, D), lambda k, j, idx: (0, 0)),
        ),
        out_shape=jax.ShapeDtypeStruct((S, D), jnp.float32),
    )(idx, probs, hn, x, fc1, b1, fc2, b2)


def kernel(x_swin, x_conv, noise_eps, params):
    p = params
    xs0 = x_swin.reshape(S, D)
    xc0 = x_conv.reshape(S, D)
    eps = noise_eps.reshape(S, E)

    xs_n, xc_n = _ln2(xs0, xc0, p['swin_pre_ln_g'], p['swin_pre_ln_b'],
                      p['conv_pre_ln_g'], p['conv_pre_ln_b'])
    xs = _mha(xs_n, xc_n, p['cross_swin_in_w'], p['cross_swin_in_b'],
              p['cross_swin_out_w'], p['cross_swin_out_b'], xs0)
    xc = _mha(xc_n, xs_n, p['cross_conv_in_w'], p['cross_conv_in_b'],
              p['cross_conv_out_w'], p['cross_conv_out_b'], xc0)
    ts, tc = _ln2(xs, xc, p['swin_self_ln_g'], p['swin_self_ln_b'],
                  p['conv_self_ln_g'], p['conv_self_ln_b'])
    xs = _mha(ts, ts, p['self_swin_in_w'], p['self_swin_in_b'],
              p['self_swin_out_w'], p['self_swin_out_b'], xs)
    xc = _mha(tc, tc, p['self_conv_in_w'], p['self_conv_in_b'],
              p['self_conv_out_w'], p['self_conv_out_b'], xc)

    rw = jnp.concatenate([p['router_w'], p['noise_w']], axis=0)
    rb = jnp.concatenate([p['router_b'], p['noise_b']], axis=0)
    x, hn, lg = _moe_prep(xs, xc, p['moe_ln_g'], p['moe_ln_b'], rw, rb)

    noisy = lg[:, :E] + eps * jax.nn.softplus(lg[:, E:])
    noisy = noisy.mean(axis=0)
    vals, idx = jax.lax.top_k(noisy, K)
    probs = jax.nn.softmax(vals).reshape(1, K)

    out = _experts(idx.astype(jnp.int32), probs, hn, x,
                   p['exp_fc1_w'], p['exp_fc1_b'].reshape(E, 1, HID),
                   p['exp_fc2_w'], p['exp_fc2_b'].reshape(E, 1, D))
    return out.reshape(B, S, D)


# speedup vs baseline: 3.6814x; 2.0145x over previous
import functools

import jax
import jax.numpy as jnp
import numpy as np
from jax.experimental import pallas as pl
from jax.experimental.pallas import tpu as pltpu

B, S, D, H, E, K = 1, 2048, 1024, 16, 8, 2
HID = 4 * D
DH = D // H
HBLK = 1024
NH = HID // HBLK
BF = jnp.bfloat16

_NN = (((1,), (0,)), ((), ()))
_TT = (((0,), (0,)), ((), ()))


def _dot(a, b):
    return jax.lax.dot_general(a.astype(BF), b.astype(BF), _NN,
                               preferred_element_type=jnp.float32)


def _lnT(xT, g, b):
    m = jnp.mean(xT, axis=0, keepdims=True)
    v = jnp.mean((xT - m) ** 2, axis=0, keepdims=True)
    return (xT - m) * jax.lax.rsqrt(v + 1e-5) * g + b


def _ln2_kernel(xs_ref, xc_ref, gs_ref, bs_ref, gc_ref, bc_ref, xn_ref):
    xn_ref[0] = _lnT(xs_ref[...], gs_ref[...], bs_ref[...]).astype(BF)
    xn_ref[1] = _lnT(xc_ref[...], gc_ref[...], bc_ref[...]).astype(BF)


def _ln2(xsT, xcT, gs, bs, gc, bc):
    return pl.pallas_call(
        _ln2_kernel,
        out_shape=jax.ShapeDtypeStruct((2, D, S), BF),
    )(xsT, xcT, gs.reshape(D, 1), bs.reshape(D, 1), gc.reshape(D, 1),
      bc.reshape(D, 1))


def _qkv_kernel(xn_ref, w_ref, b_ref, qkv_ref):
    j = pl.program_id(0)
    scale = jnp.where(j == 0, np.float32(1.0 / np.sqrt(DH)), np.float32(1.0))
    out = _dot(w_ref[...], xn_ref[0]) + b_ref[0]
    qkv_ref[...] = (out * scale).astype(BF)


def _qkv(xn, in_w, in_b, q_stream, kv_stream):
    qs, ks = q_stream, kv_stream
    if qs == ks:
        idx = lambda j: (qs, 0, 0)
    elif qs == 0:
        idx = lambda j: (jnp.minimum(j, 1), 0, 0)
    else:
        idx = lambda j: (1 - jnp.minimum(j, 1), 0, 0)
    return pl.pallas_call(
        _qkv_kernel,
        grid=(3,),
        in_specs=[
            pl.BlockSpec((1, D, S), idx),
            pl.BlockSpec((D, D), lambda j: (j, 0)),
            pl.BlockSpec((1, D, 1), lambda j: (j, 0, 0)),
        ],
        out_specs=pl.BlockSpec((D, S), lambda j: (j, 0)),
        out_shape=jax.ShapeDtypeStruct((3 * D, S), BF),
    )(xn, in_w, in_b.reshape(3, D, 1))


def _att_kernel(q_ref, k_ref, v_ref, o_ref):
    sT = jax.lax.dot_general(k_ref[...].astype(BF), q_ref[...].astype(BF),
                             _TT, preferred_element_type=jnp.float32)
    eT = jnp.exp(sT)
    rs = jnp.sum(eT, axis=0, keepdims=True)
    oT = _dot(v_ref[...], eT)
    o_ref[...] = (oT * (1.0 / rs)).astype(BF)


def _att(qkvT):
    return pl.pallas_call(
        _att_kernel,
        grid=(H,),
        in_specs=[
            pl.BlockSpec((DH, S), lambda h: (h, 0)),
            pl.BlockSpec((DH, S), lambda h: (H + h, 0)),
            pl.BlockSpec((DH, S), lambda h: (2 * H + h, 0)),
        ],
        out_specs=pl.BlockSpec((DH, S), lambda h: (h, 0)),
        out_shape=jax.ShapeDtypeStruct((D, S), BF),
    )(qkvT, qkvT, qkvT)


def _proj_kernel(o_ref, w_ref, b_ref, res_ref, out_ref):
    out_ref[...] = _dot(w_ref[...], o_ref[...]) + b_ref[...] + res_ref[...]


def _proj(oT, out_w, out_b, resT):
    return pl.pallas_call(
        _proj_kernel,
        out_shape=jax.ShapeDtypeStruct((D, S), jnp.float32),
    )(oT, out_w, out_b.reshape(D, 1), resT)


def _mha(xn, in_w, in_b, out_w, out_b, resT, q_stream, kv_stream):
    qkvT = _qkv(xn, in_w, in_b, q_stream, kv_stream)
    oT = _att(qkvT)
    return _proj(oT, out_w, out_b, resT)


def _moe_prep_kernel(xs_ref, xc_ref, g_ref, b_ref, rw_ref, rb_ref,
                     x_ref, hn_ref, lg_ref):
    x = xs_ref[...] + xc_ref[...]
    hn = _lnT(x, g_ref[...], b_ref[...])
    x_ref[...] = x
    hn_ref[...] = hn.astype(BF)
    lg_ref[...] = jax.lax.dot_general(
        rw_ref[...], hn, _NN, preferred_element_type=jnp.float32) + rb_ref[...]


def _moe_prep(xsT, xcT, g, b, rw, rb):
    return pl.pallas_call(
        _moe_prep_kernel,
        out_shape=(jax.ShapeDtypeStruct((D, S), jnp.float32),
                   jax.ShapeDtypeStruct((D, S), BF),
                   jax.ShapeDtypeStruct((2 * E, S), jnp.float32)),
    )(xsT, xcT, g.reshape(D, 1), b.reshape(D, 1), rw, rb.reshape(2 * E, 1))


def _expert_kernel(idx_ref, probs_ref, hn_ref, x_ref, fc1_ref, b1_ref,
                   fc2_ref, b2_ref, out_ref):
    ki = pl.program_id(0)
    j = pl.program_id(1)
    p = probs_ref[0, ki]
    h1 = _dot(fc1_ref[0], hn_ref[...]) + b1_ref[0]
    h1 = h1 * 0.5 * (1.0 + jax.lax.erf(h1 * np.float32(1.0 / np.sqrt(2.0))))
    part = _dot(fc2_ref[0], h1)

    @pl.when((ki == 0) & (j == 0))
    def _():
        out_ref[...] = x_ref[...]

    @pl.when(j == 0)
    def _():
        out_ref[...] += p * b2_ref[0]

    out_ref[...] += p * part


def _experts(idx, probs, hnT, xT, fc1, b1, fc2, b2):
    grid = (K, NH)
    return pl.pallas_call(
        _expert_kernel,
        grid_spec=pltpu.PrefetchScalarGridSpec(
            num_scalar_prefetch=1,
            grid=grid,
            in_specs=[
                pl.BlockSpec(memory_space=pltpu.SMEM),
                pl.BlockSpec((D, S), lambda k, j, idx: (0, 0)),
                pl.BlockSpec((D, S), lambda k, j, idx: (0, 0)),
                pl.BlockSpec((1, HBLK, D), lambda k, j, idx: (idx[k], j, 0)),
                pl.BlockSpec((1, HBLK, 1), lambda k, j, idx: (idx[k], j, 0)),
                pl.BlockSpec((1, D, HBLK), lambda k, j, idx: (idx[k], 0, j)),
                pl.BlockSpec((1, D, 1), lambda k, j, idx: (idx[k], 0, 0)),
            ],
            out_specs=pl.BlockSpec((D, S), lambda k, j, idx: (0, 0)),
        ),
        out_shape=jax.ShapeDtypeStruct((D, S), jnp.float32),
    )(idx, probs, hnT, xT, fc1, b1, fc2, b2)


def kernel(x_swin, x_conv, noise_eps, params):
    p = params
    xs0T = x_swin.reshape(S, D).T
    xc0T = x_conv.reshape(S, D).T
    epsT = noise_eps.reshape(S, E).T

    xn = _ln2(xs0T, xc0T, p['swin_pre_ln_g'], p['swin_pre_ln_b'],
              p['conv_pre_ln_g'], p['conv_pre_ln_b'])
    xsT = _mha(xn, p['cross_swin_in_w'], p['cross_swin_in_b'],
               p['cross_swin_out_w'], p['cross_swin_out_b'], xs0T, 0, 1)
    xcT = _mha(xn, p['cross_conv_in_w'], p['cross_conv_in_b'],
               p['cross_conv_out_w'], p['cross_conv_out_b'], xc0T, 1, 0)
    tn = _ln2(xsT, xcT, p['swin_self_ln_g'], p['swin_self_ln_b'],
              p['conv_self_ln_g'], p['conv_self_ln_b'])
    xsT = _mha(tn, p['self_swin_in_w'], p['self_swin_in_b'],
               p['self_swin_out_w'], p['self_swin_out_b'], xsT, 0, 0)
    xcT = _mha(tn, p['self_conv_in_w'], p['self_conv_in_b'],
               p['self_conv_out_w'], p['self_conv_out_b'], xcT, 1, 1)

    rw = jnp.concatenate([p['router_w'], p['noise_w']], axis=0)
    rb = jnp.concatenate([p['router_b'], p['noise_b']], axis=0)
    xT, hnT, lg = _moe_prep(xsT, xcT, p['moe_ln_g'], p['moe_ln_b'], rw, rb)

    noisy = lg[:E] + epsT * jax.nn.softplus(lg[E:])
    noisy = noisy.mean(axis=1)
    vals, idx = jax.lax.top_k(noisy, K)
    probs = jax.nn.softmax(vals).reshape(1, K)

    outT = _experts(idx.astype(jnp.int32), probs, hnT, xT,
                    p['exp_fc1_w'], p['exp_fc1_b'].reshape(E, HID, 1),
                    p['exp_fc2_w'], p['exp_fc2_b'].reshape(E, D, 1))
    return outT.T.reshape(B, S, D)


# speedup vs baseline: 3.8966x; 1.0585x over previous
import functools

import jax
import jax.numpy as jnp
import numpy as np
from jax.experimental import pallas as pl
from jax.experimental.pallas import tpu as pltpu

B, S, D, H, E, K = 1, 2048, 1024, 16, 8, 2
HID = 4 * D
DH = D // H
HBLK = 1024
NH = HID // HBLK
BF = jnp.bfloat16

_NN = (((1,), (0,)), ((), ()))
_TT = (((0,), (0,)), ((), ()))


def _dot(a, b):
    return jax.lax.dot_general(a.astype(BF), b.astype(BF), _NN,
                               preferred_element_type=jnp.float32)


def _lnT(xT, g, b):
    m = jnp.mean(xT, axis=0, keepdims=True)
    v = jnp.mean((xT - m) ** 2, axis=0, keepdims=True)
    return (xT - m) * jax.lax.rsqrt(v + 1e-5) * g + b


def _ln2_kernel(x_ref, gs_ref, bs_ref, gc_ref, bc_ref, xn_ref):
    xn_ref[0] = _lnT(x_ref[0], gs_ref[...], bs_ref[...]).astype(BF)
    xn_ref[1] = _lnT(x_ref[1], gc_ref[...], bc_ref[...]).astype(BF)


def _ln2(x2T, gs, bs, gc, bc):
    return pl.pallas_call(
        _ln2_kernel,
        out_shape=jax.ShapeDtypeStruct((2, D, S), BF),
    )(x2T, gs.reshape(D, 1), bs.reshape(D, 1), gc.reshape(D, 1),
      bc.reshape(D, 1))


def _qkv2_kernel(xn_ref, wa_ref, wb_ref, ba_ref, bb_ref, qkv_ref):
    g = pl.program_id(0)
    j = pl.program_id(1)
    scale = jnp.where(j == 0, np.float32(1.0 / np.sqrt(DH)), np.float32(1.0))

    @pl.when(g == 0)
    def _():
        out = _dot(wa_ref[...], xn_ref[0]) + ba_ref[0]
        qkv_ref[0] = (out * scale).astype(BF)

    @pl.when(g == 1)
    def _():
        out = _dot(wb_ref[...], xn_ref[0]) + bb_ref[0]
        qkv_ref[0] = (out * scale).astype(BF)


def _qkv2(xn, in_w_a, in_b_a, in_w_b, in_b_b, cross):
    if cross:
        xn_idx = lambda g, j: ((g + jnp.minimum(j, 1)) % 2, 0, 0)
    else:
        xn_idx = lambda g, j: (g, 0, 0)
    wa_idx = lambda g, j: (jnp.where(g == 0, j, 2), 0)
    wb_idx = lambda g, j: (jnp.where(g == 1, j, 0), 0)
    return pl.pallas_call(
        _qkv2_kernel,
        grid=(2, 3),
        in_specs=[
            pl.BlockSpec((1, D, S), xn_idx),
            pl.BlockSpec((D, D), wa_idx),
            pl.BlockSpec((D, D), wb_idx),
            pl.BlockSpec((1, D, 1), lambda g, j: (jnp.where(g == 0, j, 2), 0, 0)),
            pl.BlockSpec((1, D, 1), lambda g, j: (jnp.where(g == 1, j, 0), 0, 0)),
        ],
        out_specs=pl.BlockSpec((1, D, S), lambda g, j: (g, j, 0)),
        out_shape=jax.ShapeDtypeStruct((2, 3 * D, S), BF),
    )(xn, in_w_a, in_w_b, in_b_a.reshape(3, D, 1), in_b_b.reshape(3, D, 1))


def _att2_kernel(q_ref, k_ref, v_ref, o_ref):
    sT = jax.lax.dot_general(k_ref[0].astype(BF), q_ref[0].astype(BF),
                             _TT, preferred_element_type=jnp.float32)
    eT = jnp.exp(sT)
    rs = jnp.sum(eT, axis=0, keepdims=True)
    oT = _dot(v_ref[0], eT)
    o_ref[0] = (oT * (1.0 / rs)).astype(BF)


def _att2(qkv2):
    return pl.pallas_call(
        _att2_kernel,
        grid=(2, H),
        in_specs=[
            pl.BlockSpec((1, DH, S), lambda g, h: (g, h, 0)),
            pl.BlockSpec((1, DH, S), lambda g, h: (g, H + h, 0)),
            pl.BlockSpec((1, DH, S), lambda g, h: (g, 2 * H + h, 0)),
        ],
        out_specs=pl.BlockSpec((1, DH, S), lambda g, h: (g, h, 0)),
        out_shape=jax.ShapeDtypeStruct((2, D, S), BF),
    )(qkv2, qkv2, qkv2)


SB = 2
SBLK = S // SB


def _proj2_ln_kernel(o_ref, wa_ref, wb_ref, ba_ref, bb_ref, res_ref,
                     ga_ref, bga_ref, gb_ref, bgb_ref, x_ref, tn_ref):
    g = pl.program_id(0)

    @pl.when(g == 0)
    def _():
        out = _dot(wa_ref[...], o_ref[0]) + ba_ref[...] + res_ref[0]
        x_ref[0] = out
        tn_ref[0] = _lnT(out, ga_ref[...], bga_ref[...]).astype(BF)

    @pl.when(g == 1)
    def _():
        out = _dot(wb_ref[...], o_ref[0]) + bb_ref[...] + res_ref[0]
        x_ref[0] = out
        tn_ref[0] = _lnT(out, gb_ref[...], bgb_ref[...]).astype(BF)


def _proj2_ln(o2, wa, ba, wb, bb, res2, ga, bga, gb, bgb):
    vec = lambda g, s: (0, 0)
    return pl.pallas_call(
        _proj2_ln_kernel,
        grid=(2, SB),
        in_specs=[
            pl.BlockSpec((1, D, SBLK), lambda g, s: (g, 0, s)),
            pl.BlockSpec((D, D), vec),
            pl.BlockSpec((D, D), vec),
            pl.BlockSpec((D, 1), vec),
            pl.BlockSpec((D, 1), vec),
            pl.BlockSpec((1, D, SBLK), lambda g, s: (g, 0, s)),
            pl.BlockSpec((D, 1), vec),
            pl.BlockSpec((D, 1), vec),
            pl.BlockSpec((D, 1), vec),
            pl.BlockSpec((D, 1), vec),
        ],
        out_specs=(pl.BlockSpec((1, D, SBLK), lambda g, s: (g, 0, s)),
                   pl.BlockSpec((1, D, SBLK), lambda g, s: (g, 0, s))),
        out_shape=(jax.ShapeDtypeStruct((2, D, S), jnp.float32),
                   jax.ShapeDtypeStruct((2, D, S), BF)),
    )(o2, wa, wb, ba.reshape(D, 1), bb.reshape(D, 1), res2,
      ga.reshape(D, 1), bga.reshape(D, 1), gb.reshape(D, 1), bgb.reshape(D, 1))


def _proj2_moe_kernel(o_ref, wa_ref, wb_ref, ba_ref, bb_ref, res_ref,
                      g_ref, b_ref, rw_ref, rb_ref, eps_ref,
                      x_ref, hn_ref, nm_ref):
    sb = pl.program_id(0)
    g = pl.program_id(1)

    @pl.when(g == 0)
    def _():
        x_ref[...] = _dot(wa_ref[...], o_ref[0]) + ba_ref[...] + res_ref[0]

    @pl.when(g == 1)
    def _():
        x = x_ref[...] + _dot(wb_ref[...], o_ref[0]) + bb_ref[...] + res_ref[0]
        x_ref[...] = x
        hn = _lnT(x, g_ref[...], b_ref[...])
        hn_ref[...] = hn.astype(BF)
        lg = jax.lax.dot_general(rw_ref[...].astype(BF), hn.astype(BF), _NN,
                                 preferred_element_type=jnp.float32) + rb_ref[...]
        sp = jnp.log1p(jnp.exp(-jnp.abs(lg[E:]))) + jnp.maximum(lg[E:], 0.0)
        noisy = lg[:E] + eps_ref[...] * sp
        part = jnp.sum(noisy, axis=1, keepdims=True) * np.float32(1.0 / S)

        @pl.when(sb == 0)
        def _():
            nm_ref[...] = part

        @pl.when(sb > 0)
        def _():
            nm_ref[...] += part


def _proj2_moe(o2, wa, ba, wb, bb, res2, g, b, rw, rb, epsT):
    vec = lambda s, g: (0, 0)
    return pl.pallas_call(
        _proj2_moe_kernel,
        grid=(SB, 2),
        in_specs=[
            pl.BlockSpec((1, D, SBLK), lambda s, g: (g, 0, s)),
            pl.BlockSpec((D, D), vec),
            pl.BlockSpec((D, D), vec),
            pl.BlockSpec((D, 1), vec),
            pl.BlockSpec((D, 1), vec),
            pl.BlockSpec((1, D, SBLK), lambda s, g: (g, 0, s)),
            pl.BlockSpec((D, 1), vec),
            pl.BlockSpec((D, 1), vec),
            pl.BlockSpec((2 * E, D), vec),
            pl.BlockSpec((2 * E, 1), vec),
            pl.BlockSpec((E, SBLK), lambda s, g: (0, s)),
        ],
        out_specs=(pl.BlockSpec((D, SBLK), lambda s, g: (0, s)),
                   pl.BlockSpec((D, SBLK), lambda s, g: (0, s)),
                   pl.BlockSpec((E, 1), vec)),
        out_shape=(jax.ShapeDtypeStruct((D, S), jnp.float32),
                   jax.ShapeDtypeStruct((D, S), BF),
                   jax.ShapeDtypeStruct((E, 1), jnp.float32)),
    )(o2, wa, wb, ba.reshape(D, 1), bb.reshape(D, 1), res2,
      g.reshape(D, 1), b.reshape(D, 1), rw, rb.reshape(2 * E, 1), epsT)


def _expert_kernel(idx_ref, probs_ref, hn_ref, x_ref, fc1_ref, b1_ref,
                   fc2_ref, b2_ref, out_ref):
    ki = pl.program_id(0)
    j = pl.program_id(1)
    p = probs_ref[0, ki]
    h1 = _dot(fc1_ref[0], hn_ref[...]) + b1_ref[0]
    h1 = h1 * 0.5 * (1.0 + jax.lax.erf(h1 * np.float32(1.0 / np.sqrt(2.0))))
    part = _dot(fc2_ref[0], h1)

    @pl.when((ki == 0) & (j == 0))
    def _():
        out_ref[...] = x_ref[...]

    @pl.when(j == 0)
    def _():
        out_ref[...] += p * b2_ref[0]

    out_ref[...] += p * part


def _experts(idx, probs, hnT, xT, fc1, b1, fc2, b2):
    grid = (K, NH)
    return pl.pallas_call(
        _expert_kernel,
        grid_spec=pltpu.PrefetchScalarGridSpec(
            num_scalar_prefetch=1,
            grid=grid,
            in_specs=[
                pl.BlockSpec(memory_space=pltpu.SMEM),
                pl.BlockSpec((D, S), lambda k, j, idx: (0, 0)),
                pl.BlockSpec((D, S), lambda k, j, idx: (0, 0)),
                pl.BlockSpec((1, HBLK, D), lambda k, j, idx: (idx[k], j, 0)),
                pl.BlockSpec((1, HBLK, 1), lambda k, j, idx: (idx[k], j, 0)),
                pl.BlockSpec((1, D, HBLK), lambda k, j, idx: (idx[k], 0, j)),
                pl.BlockSpec((1, D, 1), lambda k, j, idx: (idx[k], 0, 0)),
            ],
            out_specs=pl.BlockSpec((D, S), lambda k, j, idx: (0, 0)),
        ),
        out_shape=jax.ShapeDtypeStruct((D, S), jnp.float32),
    )(idx, probs, hnT, xT, fc1, b1, fc2, b2)


def kernel(x_swin, x_conv, noise_eps, params):
    p = params
    x0 = jnp.stack([x_swin.reshape(S, D).T,
                    x_conv.reshape(S, D).T])
    epsT = noise_eps.reshape(S, E).T

    xn = _ln2(x0, p['swin_pre_ln_g'], p['swin_pre_ln_b'],
              p['conv_pre_ln_g'], p['conv_pre_ln_b'])
    qkvc = _qkv2(xn, p['cross_swin_in_w'], p['cross_swin_in_b'],
                 p['cross_conv_in_w'], p['cross_conv_in_b'], cross=True)
    oc = _att2(qkvc)
    x2, tn = _proj2_ln(oc, p['cross_swin_out_w'], p['cross_swin_out_b'],
                       p['cross_conv_out_w'], p['cross_conv_out_b'],
                       x0,
                       p['swin_self_ln_g'], p['swin_self_ln_b'],
                       p['conv_self_ln_g'], p['conv_self_ln_b'])
    qkvs = _qkv2(tn, p['self_swin_in_w'], p['self_swin_in_b'],
                 p['self_conv_in_w'], p['self_conv_in_b'], cross=False)
    os_ = _att2(qkvs)
    rw = jnp.concatenate([p['router_w'], p['noise_w']], axis=0)
    rb = jnp.concatenate([p['router_b'], p['noise_b']], axis=0)
    xT, hnT, nm = _proj2_moe(os_, p['self_swin_out_w'], p['self_swin_out_b'],
                             p['self_conv_out_w'], p['self_conv_out_b'],
                             x2, p['moe_ln_g'], p['moe_ln_b'], rw, rb, epsT)

    vals, idx = jax.lax.top_k(nm.reshape(E), K)
    probs = jax.nn.softmax(vals).reshape(1, K)

    outT = _experts(idx.astype(jnp.int32), probs, hnT, xT,
                    p['exp_fc1_w'], p['exp_fc1_b'].reshape(E, HID, 1),
                    p['exp_fc2_w'], p['exp_fc2_b'].reshape(E, D, 1))
    return outT.T.reshape(B, S, D)
